# Initial kernel scaffold; baseline (speedup 1.0000x reference)
import jax
import jax.numpy as jnp
from jax.experimental import pallas as pl


def kernel(z, emb_weight, W, b):
    raise NotImplementedError("write your pallas kernel here")


# speedup vs baseline: 1.8994x; 1.8994x over previous
import functools

import jax
import jax.numpy as jnp
from jax import lax
from jax.experimental import pallas as pl
from jax.experimental.pallas import tpu as pltpu
from jax.experimental.pallas import tpu_sc as plsc

N = 8192
K = 8192
D = 64
TN = 512
TK = 2048
NI = N // TN
NJ = K // TK

_PREC = lax.Precision.DEFAULT


def _codebook_body(emb_ref, w_ref, b_ref, out_ref):
    out_ref[...] = lax.dot_general(
        emb_ref[...], w_ref[...], (((1,), (1,)), ((), ())),
        precision=_PREC, preferred_element_type=jnp.float32) + b_ref[...]


_codebook_call = pl.pallas_call(
    _codebook_body,
    out_shape=jax.ShapeDtypeStruct((K, D), jnp.float32),
)


def _dist_body(z_ref, c_ref, d_ref, idx_ref, bval_ref, bidx_ref):
    j = pl.program_id(1)
    z = z_ref[...]
    c = c_ref[...]
    z2 = jnp.sum(z * z, axis=1, keepdims=True)
    c2 = jnp.sum(c * c, axis=1)
    dot = lax.dot_general(z, c, (((1,), (1,)), ((), ())),
                          precision=_PREC, preferred_element_type=jnp.float32)
    dist = z2 + c2[None, :] - 2.0 * dot
    d_ref[...] = dist
    row_min = jnp.min(dist, axis=1)
    ids = lax.broadcasted_iota(jnp.int32, (TN, TK), 1)
    masked = jnp.where(dist == row_min[:, None], ids, TK)
    loc = jnp.min(masked, axis=1) + j * TK

    @pl.when(j == 0)
    def _():
        bval_ref[...] = row_min
        bidx_ref[...] = loc

    @pl.when(j > 0)
    def _():
        better = row_min < bval_ref[...]
        bval_ref[...] = jnp.where(better, row_min, bval_ref[...])
        bidx_ref[...] = jnp.where(better, loc, bidx_ref[...])

    @pl.when(j == pl.num_programs(1) - 1)
    def _():
        idx_ref[...] = bidx_ref[...].reshape(1, 1, TN)


_dist_call = pl.pallas_call(
    _dist_body,
    grid=(NI, NJ),
    in_specs=[
        pl.BlockSpec((TN, D), lambda i, j: (i, 0)),
        pl.BlockSpec((TK, D), lambda i, j: (j, 0)),
    ],
    out_specs=[
        pl.BlockSpec((TN, TK), lambda i, j: (i, j)),
        pl.BlockSpec((1, 1, TN), lambda i, j: (i, 0, 0)),
    ],
    out_shape=[
        jax.ShapeDtypeStruct((N, K), jnp.float32),
        jax.ShapeDtypeStruct((NI, 1, TN), jnp.int32),
    ],
    scratch_shapes=[
        pltpu.VMEM((TN,), jnp.float32),
        pltpu.VMEM((TN,), jnp.int32),
    ],
)


_SC_NC = 2
_SC_NS = 16
_NW = _SC_NC * _SC_NS
_BPW = N // _NW


def _gather_body(table_hbm, idx_hbm, out_hbm, idx_v, rows_v, sem):
    wid = lax.axis_index("s") * _SC_NC + lax.axis_index("c")
    base = wid * _BPW
    pltpu.sync_copy(idx_hbm.at[pl.ds(base, _BPW)], idx_v)
    pltpu.async_copy(table_hbm.at[idx_v], rows_v, sem).wait()
    pltpu.sync_copy(rows_v, out_hbm.at[pl.ds(base, _BPW)])


@functools.cache
def _get_gather_call():
    return pl.kernel(
        _gather_body,
        mesh=plsc.VectorSubcoreMesh(core_axis_name="c", subcore_axis_name="s"),
        out_type=jax.ShapeDtypeStruct((N, D), jnp.float32),
        scratch_types=[
            pltpu.VMEM((_BPW,), jnp.int32),
            pltpu.VMEM((_BPW, D), jnp.float32),
            pltpu.SemaphoreType.DMA,
        ],
        compiler_params=pltpu.CompilerParams(use_tc_tiling_on_sc=False),
    )


_HN = 512
_HK = 2048


def _stats_body(z_ref, q_ref, idx_ref, zst_ref, loss_ref, ppl_ref):
    z = z_ref[...]
    q = q_ref[...]
    diff = q - z
    zst_ref[...] = z + diff
    m = jnp.mean(diff * diff)
    loss_ref[...] = jnp.reshape(1.0 * m + 0.25 * m, (1, 1))

    def body(r, acc):
        row = idx_ref[r]
        rowc = jnp.broadcast_to(row.reshape(_HN, 1), (_HN, _HK))
        parts = []
        for cb in range(K // _HK):
            bins = lax.broadcasted_iota(jnp.int32, (_HN, _HK), 1) + cb * _HK
            cmp = (rowc == bins).astype(jnp.float32)
            parts.append(jnp.sum(cmp, axis=0))
        return acc + jnp.stack(parts)

    counts = lax.fori_loop(0, N // _HN, body, jnp.zeros((K // _HK, _HK), jnp.float32))
    e = counts * (1.0 / N)
    ent = jnp.sum(e * jnp.log(e + 1e-10))
    ppl_ref[...] = jnp.reshape(jnp.exp(-ent), (1, 1))


_stats_call = pl.pallas_call(
    _stats_body,
    out_shape=[
        jax.ShapeDtypeStruct((N, D), jnp.float32),
        jax.ShapeDtypeStruct((1, 1), jnp.float32),
        jax.ShapeDtypeStruct((1, 1), jnp.float32),
    ],
)


def kernel(z, emb_weight, W, b):
    zt = jnp.transpose(z, (0, 2, 3, 1))
    z_flat = zt.reshape(-1, D)
    codebook = _codebook_call(emb_weight, W, b.reshape(1, D))
    d, idx3 = _dist_call(z_flat, codebook)
    idx = idx3.reshape(N)
    z_q = _get_gather_call()(codebook, idx)
    z_q_st, loss11, ppl11 = _stats_call(z_flat, z_q, idx.reshape(N // _HN, _HN))
    z_q_out = jnp.transpose(z_q_st.reshape(zt.shape), (0, 3, 1, 2))
    loss = loss11[0, 0]
    perplexity = ppl11[0, 0]
    perplexity_loss = jnp.zeros((), jnp.float32)
    return (z_q_out, loss, d, perplexity, idx, perplexity_loss)


# speedup vs baseline: 1.9985x; 1.0522x over previous
import functools

import jax
import jax.numpy as jnp
from jax import lax
from jax.experimental import pallas as pl
from jax.experimental.pallas import tpu as pltpu
from jax.experimental.pallas import tpu_sc as plsc

N = 8192
K = 8192
D = 64
TN = 512
TK = 2048
NI = N // TN
NJ = K // TK

_PREC = lax.Precision.DEFAULT


def _codebook_body(emb_ref, w_ref, b_ref, out_ref, c2_ref):
    cb = lax.dot_general(
        emb_ref[...], w_ref[...], (((1,), (1,)), ((), ())),
        precision=_PREC, preferred_element_type=jnp.float32) + b_ref[...]
    out_ref[...] = cb
    c2_ref[...] = jnp.sum(cb * cb, axis=1).reshape(1, K)


_codebook_call = pl.pallas_call(
    _codebook_body,
    out_shape=[
        jax.ShapeDtypeStruct((K, D), jnp.float32),
        jax.ShapeDtypeStruct((1, K), jnp.float32),
    ],
)


def _dist_body(z_ref, c_ref, c2_ref, ids_ref, d_ref, idx_ref, bval_ref, bidx_ref):
    j = pl.program_id(1)
    z = z_ref[...]
    c = c_ref[...]
    z2 = jnp.sum(z * z, axis=1, keepdims=True)
    c2 = c2_ref[...]
    dot = lax.dot_general(z, c, (((1,), (1,)), ((), ())),
                          precision=_PREC, preferred_element_type=jnp.float32)
    dist = z2 + c2 - 2.0 * dot
    d_ref[...] = dist
    row_min = jnp.min(dist, axis=1)
    masked = jnp.where(dist == row_min[:, None], ids_ref[...], jnp.float32(2 * K))
    loc = jnp.min(masked, axis=1).astype(jnp.int32)

    @pl.when(j == 0)
    def _():
        bval_ref[...] = row_min
        bidx_ref[...] = loc

    @pl.when(j > 0)
    def _():
        better = row_min < bval_ref[...]
        bval_ref[...] = jnp.where(better, row_min, bval_ref[...])
        bidx_ref[...] = jnp.where(better, loc, bidx_ref[...])

    @pl.when(j == pl.num_programs(1) - 1)
    def _():
        idx_ref[...] = bidx_ref[...].reshape(1, 1, TN)


_dist_call = pl.pallas_call(
    _dist_body,
    grid=(NI, NJ),
    in_specs=[
        pl.BlockSpec((TN, D), lambda i, j: (i, 0)),
        pl.BlockSpec((TK, D), lambda i, j: (j, 0)),
        pl.BlockSpec((1, TK), lambda i, j: (0, j)),
        pl.BlockSpec((1, TK), lambda i, j: (0, j)),
    ],
    out_specs=[
        pl.BlockSpec((TN, TK), lambda i, j: (i, j)),
        pl.BlockSpec((1, 1, TN), lambda i, j: (i, 0, 0)),
    ],
    out_shape=[
        jax.ShapeDtypeStruct((N, K), jnp.float32),
        jax.ShapeDtypeStruct((NI, 1, TN), jnp.int32),
    ],
    scratch_shapes=[
        pltpu.VMEM((TN,), jnp.float32),
        pltpu.VMEM((TN,), jnp.int32),
    ],
)


_SC_NC = 2
_SC_NS = 16
_NW = _SC_NC * _SC_NS
_BPW = N // _NW


def _gather_body(table_hbm, idx_hbm, out_hbm, idx_v, rows_v, sem):
    wid = lax.axis_index("s") * _SC_NC + lax.axis_index("c")
    base = wid * _BPW
    pltpu.sync_copy(idx_hbm.at[pl.ds(base, _BPW)], idx_v)
    pltpu.async_copy(table_hbm.at[idx_v], rows_v, sem).wait()
    pltpu.sync_copy(rows_v, out_hbm.at[pl.ds(base, _BPW)])


@functools.cache
def _get_gather_call():
    return pl.kernel(
        _gather_body,
        mesh=plsc.VectorSubcoreMesh(core_axis_name="c", subcore_axis_name="s"),
        out_type=jax.ShapeDtypeStruct((N, D), jnp.float32),
        scratch_types=[
            pltpu.VMEM((_BPW,), jnp.int32),
            pltpu.VMEM((_BPW, D), jnp.float32),
            pltpu.SemaphoreType.DMA,
        ],
        compiler_params=pltpu.CompilerParams(use_tc_tiling_on_sc=False),
    )


_HN = 512
_HK = 2048


def _stats_body(z_ref, q_ref, idx_ref, zst_ref, loss_ref, ppl_ref):
    z = z_ref[...]
    q = q_ref[...]
    diff = q - z
    zst_ref[...] = z + diff
    m = jnp.mean(diff * diff)
    loss_ref[...] = jnp.reshape(1.0 * m + 0.25 * m, (1, 1))

    def body(r, acc):
        row = idx_ref[r]
        rowc = jnp.broadcast_to(row.reshape(_HN, 1), (_HN, _HK))
        parts = []
        for cb in range(K // _HK):
            bins = lax.broadcasted_iota(jnp.int32, (_HN, _HK), 1) + cb * _HK
            cmp = (rowc == bins).astype(jnp.float32)
            parts.append(jnp.sum(cmp, axis=0))
        return acc + jnp.stack(parts)

    counts = lax.fori_loop(0, N // _HN, body, jnp.zeros((K // _HK, _HK), jnp.float32))
    e = counts * (1.0 / N)
    ent = jnp.sum(e * jnp.log(e + 1e-10))
    ppl_ref[...] = jnp.reshape(jnp.exp(-ent), (1, 1))


_stats_call = pl.pallas_call(
    _stats_body,
    out_shape=[
        jax.ShapeDtypeStruct((N, D), jnp.float32),
        jax.ShapeDtypeStruct((1, 1), jnp.float32),
        jax.ShapeDtypeStruct((1, 1), jnp.float32),
    ],
)


def kernel(z, emb_weight, W, b):
    zt = jnp.transpose(z, (0, 2, 3, 1))
    z_flat = zt.reshape(-1, D)
    codebook, c2 = _codebook_call(emb_weight, W, b.reshape(1, D))
    ids_f = jnp.arange(K, dtype=jnp.float32).reshape(1, K)
    d, idx3 = _dist_call(z_flat, codebook, c2, ids_f)
    idx = idx3.reshape(N)
    z_q = _get_gather_call()(codebook, idx)
    z_q_st, loss11, ppl11 = _stats_call(z_flat, z_q, idx.reshape(N // _HN, _HN))
    z_q_out = jnp.transpose(z_q_st.reshape(zt.shape), (0, 3, 1, 2))
    loss = loss11[0, 0]
    perplexity = ppl11[0, 0]
    perplexity_loss = jnp.zeros((), jnp.float32)
    return (z_q_out, loss, d, perplexity, idx, perplexity_loss)


# speedup vs baseline: 2.5139x; 1.2579x over previous
import functools

import jax
import jax.numpy as jnp
from jax import lax
from jax.experimental import pallas as pl
from jax.experimental.pallas import tpu as pltpu
from jax.experimental.pallas import tpu_sc as plsc

N = 8192
K = 8192
D = 64
TN = 128
TK = K
NI = N // TN

_PREC = lax.Precision.DEFAULT


def _codebook_body(emb_ref, w_ref, b_ref, out_ref, c2_ref):
    cb = lax.dot_general(
        emb_ref[...], w_ref[...], (((1,), (1,)), ((), ())),
        precision=_PREC, preferred_element_type=jnp.float32) + b_ref[...]
    out_ref[...] = cb
    c2_ref[...] = jnp.sum(cb * cb, axis=1).reshape(1, K)


_codebook_call = pl.pallas_call(
    _codebook_body,
    out_shape=[
        jax.ShapeDtypeStruct((K, D), jnp.float32),
        jax.ShapeDtypeStruct((1, K), jnp.float32),
    ],
)


def _dist_body(z_ref, c_ref, c2_ref, ids_ref, d_ref, idx_ref):
    z = z_ref[...]
    c = c_ref[...]
    z2 = jnp.sum(z * z, axis=1, keepdims=True)
    c2 = c2_ref[...]
    dot = lax.dot_general(z, c, (((1,), (1,)), ((), ())),
                          precision=_PREC, preferred_element_type=jnp.float32)
    dist = z2 + c2 - 2.0 * dot
    d_ref[...] = dist
    row_min = jnp.min(dist, axis=1)
    masked = jnp.where(dist == row_min[:, None], ids_ref[...], jnp.float32(2 * K))
    idx_ref[...] = jnp.min(masked, axis=1).astype(jnp.int32).reshape(1, 1, TN)


_dist_call = pl.pallas_call(
    _dist_body,
    grid=(NI,),
    in_specs=[
        pl.BlockSpec((TN, D), lambda i: (i, 0)),
        pl.BlockSpec((K, D), lambda i: (0, 0)),
        pl.BlockSpec((1, K), lambda i: (0, 0)),
        pl.BlockSpec((1, K), lambda i: (0, 0)),
    ],
    out_specs=[
        pl.BlockSpec((TN, K), lambda i: (i, 0)),
        pl.BlockSpec((1, 1, TN), lambda i: (i, 0, 0)),
    ],
    out_shape=[
        jax.ShapeDtypeStruct((N, K), jnp.float32),
        jax.ShapeDtypeStruct((NI, 1, TN), jnp.int32),
    ],
)


_SC_NC = 2
_SC_NS = 16
_NW = _SC_NC * _SC_NS
_BPW = N // _NW


def _gather_body(table_hbm, idx_hbm, out_hbm, idx_v, rows_v, sem):
    wid = lax.axis_index("s") * _SC_NC + lax.axis_index("c")
    base = wid * _BPW
    pltpu.sync_copy(idx_hbm.at[pl.ds(base, _BPW)], idx_v)
    pltpu.async_copy(table_hbm.at[idx_v], rows_v, sem).wait()
    pltpu.sync_copy(rows_v, out_hbm.at[pl.ds(base, _BPW)])


@functools.cache
def _get_gather_call():
    return pl.kernel(
        _gather_body,
        mesh=plsc.VectorSubcoreMesh(core_axis_name="c", subcore_axis_name="s"),
        out_type=jax.ShapeDtypeStruct((N, D), jnp.float32),
        scratch_types=[
            pltpu.VMEM((_BPW,), jnp.int32),
            pltpu.VMEM((_BPW, D), jnp.float32),
            pltpu.SemaphoreType.DMA,
        ],
        compiler_params=pltpu.CompilerParams(use_tc_tiling_on_sc=False),
    )


_HN = 512
_HK = 2048


def _stats_body(z_ref, q_ref, idx_ref, zst_ref, loss_ref, ppl_ref):
    z = z_ref[...]
    q = q_ref[...]
    diff = q - z
    zst_ref[...] = z + diff
    m = jnp.mean(diff * diff)
    loss_ref[...] = jnp.reshape(1.0 * m + 0.25 * m, (1, 1))

    def body(r, acc):
        row = idx_ref[r]
        rowc = jnp.broadcast_to(row.reshape(_HN, 1), (_HN, _HK))
        parts = []
        for cb in range(K // _HK):
            bins = lax.broadcasted_iota(jnp.int32, (_HN, _HK), 1) + cb * _HK
            cmp = (rowc == bins).astype(jnp.float32)
            parts.append(jnp.sum(cmp, axis=0))
        return acc + jnp.stack(parts)

    counts = lax.fori_loop(0, N // _HN, body, jnp.zeros((K // _HK, _HK), jnp.float32))
    e = counts * (1.0 / N)
    ent = jnp.sum(e * jnp.log(e + 1e-10))
    ppl_ref[...] = jnp.reshape(jnp.exp(-ent), (1, 1))


_stats_call = pl.pallas_call(
    _stats_body,
    out_shape=[
        jax.ShapeDtypeStruct((N, D), jnp.float32),
        jax.ShapeDtypeStruct((1, 1), jnp.float32),
        jax.ShapeDtypeStruct((1, 1), jnp.float32),
    ],
)


def kernel(z, emb_weight, W, b):
    zt = jnp.transpose(z, (0, 2, 3, 1))
    z_flat = zt.reshape(-1, D)
    codebook, c2 = _codebook_call(emb_weight, W, b.reshape(1, D))
    ids_f = jnp.arange(K, dtype=jnp.float32).reshape(1, K)
    d, idx3 = _dist_call(z_flat, codebook, c2, ids_f)
    idx = idx3.reshape(N)
    z_q = _get_gather_call()(codebook, idx)
    z_q_st, loss11, ppl11 = _stats_call(z_flat, z_q, idx.reshape(N // _HN, _HN))
    z_q_out = jnp.transpose(z_q_st.reshape(zt.shape), (0, 3, 1, 2))
    loss = loss11[0, 0]
    perplexity = ppl11[0, 0]
    perplexity_loss = jnp.zeros((), jnp.float32)
    return (z_q_out, loss, d, perplexity, idx, perplexity_loss)


# speedup vs baseline: 2.9127x; 1.1586x over previous
import functools

import jax
import jax.numpy as jnp
from jax import lax
from jax.experimental import pallas as pl
from jax.experimental.pallas import tpu as pltpu
from jax.experimental.pallas import tpu_sc as plsc

N = 8192
K = 8192
D = 64
TN = 128
TK = K
NI = N // TN

_PREC = lax.Precision.DEFAULT


def _codebook_body(emb_ref, w_ref, b_ref, out_ref, c2_ref):
    cb = lax.dot_general(
        emb_ref[...], w_ref[...], (((1,), (1,)), ((), ())),
        precision=_PREC, preferred_element_type=jnp.float32) + b_ref[...]
    out_ref[...] = cb
    c2_ref[...] = jnp.sum(cb * cb, axis=1).reshape(1, K)


_codebook_call = pl.pallas_call(
    _codebook_body,
    out_shape=[
        jax.ShapeDtypeStruct((K, D), jnp.float32),
        jax.ShapeDtypeStruct((1, K), jnp.float32),
    ],
)


def _dist_body(z_ref, c_ref, c2_ref, ids_ref, d_ref, idx_ref):
    z = z_ref[...]
    c = c_ref[...]
    z2 = jnp.sum(z * z, axis=1, keepdims=True)
    c2 = c2_ref[...]
    dot = lax.dot_general(z, c, (((1,), (1,)), ((), ())),
                          precision=_PREC, preferred_element_type=jnp.float32)
    dist = z2 + c2 - 2.0 * dot
    d_ref[...] = dist
    row_min = jnp.min(dist, axis=1)
    masked = jnp.where(dist == row_min[:, None], ids_ref[...], jnp.float32(2 * K))
    idx_ref[...] = jnp.min(masked, axis=1).astype(jnp.int32).reshape(1, 1, TN)


_dist_call = pl.pallas_call(
    _dist_body,
    grid=(NI,),
    in_specs=[
        pl.BlockSpec((TN, D), lambda i: (i, 0)),
        pl.BlockSpec((K, D), lambda i: (0, 0)),
        pl.BlockSpec((1, K), lambda i: (0, 0)),
        pl.BlockSpec((1, K), lambda i: (0, 0)),
    ],
    out_specs=[
        pl.BlockSpec((TN, K), lambda i: (i, 0)),
        pl.BlockSpec((1, 1, TN), lambda i: (i, 0, 0)),
    ],
    out_shape=[
        jax.ShapeDtypeStruct((N, K), jnp.float32),
        jax.ShapeDtypeStruct((NI, 1, TN), jnp.int32),
    ],
)


_SC_NC = 2
_SC_NS = 16
_NW = _SC_NC * _SC_NS
_BPW = N // _NW


def _gather_body(table_hbm, idx_hbm, z_hbm, out_hbm, loss_hbm, cnt_hbm,
                 idx_v, rows_v, z_v, ones_v, zeros_v, acc_v, shared_cnt, sem):
    cid = lax.axis_index("c")
    sid = lax.axis_index("s")
    wid = sid * _SC_NC + cid
    base = wid * _BPW
    pltpu.sync_copy(idx_hbm.at[pl.ds(base, _BPW)], idx_v)
    gather = pltpu.async_copy(table_hbm.at[idx_v], rows_v, sem)
    pltpu.sync_copy(z_hbm.at[pl.ds(base, _BPW)], z_v)

    def zfill(i, _):
        zeros_v[pl.ds(i * 16, 16)] = jnp.zeros((16,), jnp.float32)
        return 0

    def ofill(i, _):
        ones_v[pl.ds(i * 16, 16)] = jnp.ones((16,), jnp.float32)
        return 0

    lax.fori_loop(0, K // 16, zfill, 0)
    lax.fori_loop(0, _BPW // 16, ofill, 0)

    @pl.when(sid == 0)
    def _():
        pltpu.sync_copy(zeros_v, shared_cnt)

    gather.wait()

    def body(r, acc):
        out = acc
        for cchunk in range(D // 16):
            sl = pl.ds(cchunk * 16, 16)
            q = rows_v[r, sl]
            zv = z_v[r, sl]
            diff = q - zv
            rows_v[r, sl] = zv + diff
            out = out + diff * diff
        return out

    acc = lax.fori_loop(0, _BPW, body, jnp.zeros((16,), jnp.float32))
    acc_v[...] = acc
    pltpu.sync_copy(rows_v, out_hbm.at[pl.ds(base, _BPW)])
    pltpu.sync_copy(acc_v, loss_hbm.at[wid])

    plsc.subcore_barrier()
    pltpu.sync_copy(ones_v, shared_cnt.at[idx_v], add=True)
    plsc.subcore_barrier()

    @pl.when(sid == 0)
    def _():
        pltpu.sync_copy(shared_cnt, cnt_hbm.at[cid])


@functools.cache
def _get_gather_call():
    return pl.kernel(
        _gather_body,
        mesh=plsc.VectorSubcoreMesh(core_axis_name="c", subcore_axis_name="s"),
        out_type=[
            jax.ShapeDtypeStruct((N, D), jnp.float32),
            jax.ShapeDtypeStruct((_NW, 16), jnp.float32),
            jax.ShapeDtypeStruct((_SC_NC, K), jnp.float32),
        ],
        scratch_types=[
            pltpu.VMEM((_BPW,), jnp.int32),
            pltpu.VMEM((_BPW, D), jnp.float32),
            pltpu.VMEM((_BPW, D), jnp.float32),
            pltpu.VMEM((_BPW,), jnp.float32),
            pltpu.VMEM((K,), jnp.float32),
            pltpu.VMEM((16,), jnp.float32),
            pltpu.VMEM_SHARED((K,), jnp.float32),
            pltpu.SemaphoreType.DMA,
        ],
        compiler_params=pltpu.CompilerParams(use_tc_tiling_on_sc=False),
    )


def _stats_body(lp_ref, cnt_ref, loss_ref, ppl_ref):
    m = jnp.sum(lp_ref[...]) * (1.0 / (N * D))
    loss_ref[...] = jnp.reshape(1.0 * m + 0.25 * m, (1, 1))
    cnt = cnt_ref[...]
    e = (cnt[0:1, :] + cnt[1:2, :]) * (1.0 / N)
    ent = jnp.sum(e * jnp.log(e + 1e-10))
    ppl_ref[...] = jnp.reshape(jnp.exp(-ent), (1, 1))


_stats_call = pl.pallas_call(
    _stats_body,
    out_shape=[
        jax.ShapeDtypeStruct((1, 1), jnp.float32),
        jax.ShapeDtypeStruct((1, 1), jnp.float32),
    ],
)


def kernel(z, emb_weight, W, b):
    zt = jnp.transpose(z, (0, 2, 3, 1))
    z_flat = zt.reshape(-1, D)
    codebook, c2 = _codebook_call(emb_weight, W, b.reshape(1, D))
    ids_f = jnp.arange(K, dtype=jnp.float32).reshape(1, K)
    d, idx3 = _dist_call(z_flat, codebook, c2, ids_f)
    idx = idx3.reshape(N)
    z_q_st, loss_parts, counts = _get_gather_call()(codebook, idx, z_flat)
    loss11, ppl11 = _stats_call(loss_parts, counts)
    z_q_out = jnp.transpose(z_q_st.reshape(zt.shape), (0, 3, 1, 2))
    loss = loss11[0, 0]
    perplexity = ppl11[0, 0]
    perplexity_loss = jnp.zeros((), jnp.float32)
    return (z_q_out, loss, d, perplexity, idx, perplexity_loss)


# speedup vs baseline: 2.9602x; 1.0163x over previous
import functools

import jax
import jax.numpy as jnp
from jax import lax
from jax.experimental import pallas as pl
from jax.experimental.pallas import tpu as pltpu
from jax.experimental.pallas import tpu_sc as plsc

N = 8192
K = 8192
D = 64
TN = 128
TK = K
NI = N // TN

_PREC = lax.Precision.DEFAULT


def _codebook_body(emb_ref, w_ref, b_ref, out_ref, c2_ref):
    cb = lax.dot_general(
        emb_ref[...], w_ref[...], (((1,), (1,)), ((), ())),
        precision=_PREC, preferred_element_type=jnp.float32) + b_ref[...]
    out_ref[...] = cb
    c2_ref[...] = jnp.sum(cb * cb, axis=1).reshape(1, K)


_codebook_call = pl.pallas_call(
    _codebook_body,
    out_shape=[
        jax.ShapeDtypeStruct((K, D), jnp.float32),
        jax.ShapeDtypeStruct((1, K), jnp.float32),
    ],
)


def _dist_body(z_ref, c_ref, c2_ref, ids_ref, d_ref, idx_ref):
    z = z_ref[...]
    c = c_ref[...]
    z2 = jnp.sum(z * z, axis=1, keepdims=True)
    c2 = c2_ref[...]
    dot = lax.dot_general(z, c, (((1,), (1,)), ((), ())),
                          precision=_PREC, preferred_element_type=jnp.float32)
    dist = z2 + c2 - 2.0 * dot
    d_ref[...] = dist
    row_min = jnp.min(dist, axis=1)
    masked = jnp.where(dist == row_min[:, None], ids_ref[...], jnp.float32(2 * K))
    idx_ref[...] = jnp.min(masked, axis=1).astype(jnp.int32).reshape(1, 1, TN)


_dist_call = pl.pallas_call(
    _dist_body,
    grid=(NI,),
    in_specs=[
        pl.BlockSpec((TN, D), lambda i: (i, 0)),
        pl.BlockSpec((K, D), lambda i: (0, 0)),
        pl.BlockSpec((1, K), lambda i: (0, 0)),
        pl.BlockSpec((1, K), lambda i: (0, 0)),
    ],
    out_specs=[
        pl.BlockSpec((TN, K), lambda i: (i, 0)),
        pl.BlockSpec((1, 1, TN), lambda i: (i, 0, 0)),
    ],
    out_shape=[
        jax.ShapeDtypeStruct((N, K), jnp.float32),
        jax.ShapeDtypeStruct((NI, 1, TN), jnp.int32),
    ],
)


_SC_NC = 2
_SC_NS = 16
_NW = _SC_NC * _SC_NS
_BPW = N // _NW


def _gather_body(table_hbm, idx_hbm, out_hbm, cnt_hbm,
                 idx_v, rows_v, ones_v, zeros_v, shared_cnt, sem):
    cid = lax.axis_index("c")
    sid = lax.axis_index("s")
    wid = sid * _SC_NC + cid
    base = wid * _BPW
    pltpu.sync_copy(idx_hbm.at[pl.ds(base, _BPW)], idx_v)
    gather = pltpu.async_copy(table_hbm.at[idx_v], rows_v, sem)

    def ofill(i, _):
        ones_v[pl.ds(i * 16, 16)] = jnp.ones((16,), jnp.float32)
        return 0

    lax.fori_loop(0, _BPW // 16, ofill, 0)

    @pl.when(sid == 0)
    def _():
        def zfill(i, _):
            zeros_v[pl.ds(i * 16, 16)] = jnp.zeros((16,), jnp.float32)
            return 0

        lax.fori_loop(0, K // 16, zfill, 0)
        pltpu.sync_copy(zeros_v, shared_cnt)

    plsc.subcore_barrier()
    pltpu.sync_copy(ones_v, shared_cnt.at[idx_v], add=True)
    gather.wait()
    pltpu.sync_copy(rows_v, out_hbm.at[pl.ds(base, _BPW)])
    plsc.subcore_barrier()

    @pl.when(sid == 0)
    def _():
        pltpu.sync_copy(shared_cnt, cnt_hbm.at[cid])


@functools.cache
def _get_gather_call():
    return pl.kernel(
        _gather_body,
        mesh=plsc.VectorSubcoreMesh(core_axis_name="c", subcore_axis_name="s"),
        out_type=[
            jax.ShapeDtypeStruct((N, D), jnp.float32),
            jax.ShapeDtypeStruct((_SC_NC, K), jnp.float32),
        ],
        scratch_types=[
            pltpu.VMEM((_BPW,), jnp.int32),
            pltpu.VMEM((_BPW, D), jnp.float32),
            pltpu.VMEM((_BPW,), jnp.float32),
            pltpu.VMEM((K,), jnp.float32),
            pltpu.VMEM_SHARED((K,), jnp.float32),
            pltpu.SemaphoreType.DMA,
        ],
        compiler_params=pltpu.CompilerParams(use_tc_tiling_on_sc=False),
    )


def _stats_body(z_ref, q_ref, cnt_ref, loss_ref, ppl_ref):
    diff = q_ref[...] - z_ref[...]
    m = jnp.mean(diff * diff)
    loss_ref[...] = jnp.reshape(1.0 * m + 0.25 * m, (1, 1))
    cnt = cnt_ref[...]
    e = (cnt[0:1, :] + cnt[1:2, :]) * (1.0 / N)
    ent = jnp.sum(e * jnp.log(e + 1e-10))
    ppl_ref[...] = jnp.reshape(jnp.exp(-ent), (1, 1))


_stats_call = pl.pallas_call(
    _stats_body,
    out_shape=[
        jax.ShapeDtypeStruct((1, 1), jnp.float32),
        jax.ShapeDtypeStruct((1, 1), jnp.float32),
    ],
)


def kernel(z, emb_weight, W, b):
    zt = jnp.transpose(z, (0, 2, 3, 1))
    z_flat = zt.reshape(-1, D)
    codebook, c2 = _codebook_call(emb_weight, W, b.reshape(1, D))
    ids_f = jnp.arange(K, dtype=jnp.float32).reshape(1, K)
    d, idx3 = _dist_call(z_flat, codebook, c2, ids_f)
    idx = idx3.reshape(N)
    z_q_st, counts = _get_gather_call()(codebook, idx)
    loss11, ppl11 = _stats_call(z_flat, z_q_st, counts)
    z_q_out = jnp.transpose(z_q_st.reshape(zt.shape), (0, 3, 1, 2))
    loss = loss11[0, 0]
    perplexity = ppl11[0, 0]
    perplexity_loss = jnp.zeros((), jnp.float32)
    return (z_q_out, loss, d, perplexity, idx, perplexity_loss)


# speedup vs baseline: 3.2671x; 1.1037x over previous
import functools

import jax
import jax.numpy as jnp
from jax import lax
from jax.experimental import pallas as pl
from jax.experimental.pallas import tpu as pltpu
from jax.experimental.pallas import tpu_sc as plsc

N = 8192
K = 8192
D = 64
TN = 256
TK = K
NI = N // TN

_PREC = lax.Precision.DEFAULT


def _codebook_body(emb_ref, w_ref, b_ref, out_ref, c2_ref):
    cb = lax.dot_general(
        emb_ref[...], w_ref[...], (((1,), (1,)), ((), ())),
        precision=_PREC, preferred_element_type=jnp.float32) + b_ref[...]
    out_ref[...] = cb
    c2_ref[...] = jnp.sum(cb * cb, axis=1).reshape(1, K)


_codebook_call = pl.pallas_call(
    _codebook_body,
    out_shape=[
        jax.ShapeDtypeStruct((K, D), jnp.float32),
        jax.ShapeDtypeStruct((1, K), jnp.float32),
    ],
)


def _dist_body(z_ref, c_ref, c2_ref, ids_ref, d_ref, idx_ref):
    z = z_ref[...]
    c = c_ref[...]
    z2 = jnp.sum(z * z, axis=1, keepdims=True)
    c2 = c2_ref[...]
    dot = lax.dot_general(z, c, (((1,), (1,)), ((), ())),
                          precision=_PREC, preferred_element_type=jnp.float32)
    dist = z2 + c2 - 2.0 * dot
    d_ref[...] = dist
    row_min = jnp.min(dist, axis=1)
    masked = jnp.where(dist == row_min[:, None], ids_ref[...], jnp.float32(2 * K))
    idx_ref[...] = jnp.min(masked, axis=1).astype(jnp.int32).reshape(1, 1, TN)


_dist_call = pl.pallas_call(
    _dist_body,
    grid=(NI,),
    in_specs=[
        pl.BlockSpec((TN, D), lambda i: (i, 0)),
        pl.BlockSpec((K, D), lambda i: (0, 0)),
        pl.BlockSpec((1, K), lambda i: (0, 0)),
        pl.BlockSpec((1, K), lambda i: (0, 0)),
    ],
    out_specs=[
        pl.BlockSpec((TN, K), lambda i: (i, 0)),
        pl.BlockSpec((1, 1, TN), lambda i: (i, 0, 0)),
    ],
    out_shape=[
        jax.ShapeDtypeStruct((N, K), jnp.float32),
        jax.ShapeDtypeStruct((NI, 1, TN), jnp.int32),
    ],
)


_SC_NC = 2
_SC_NS = 16
_NW = _SC_NC * _SC_NS
_BPW = N // _NW


def _gather_body(table_hbm, idx_hbm, zeros_hbm, out_hbm, cnt_hbm,
                 idx_v, rows_v, ones_v, shared_cnt, sem):
    cid = lax.axis_index("c")
    sid = lax.axis_index("s")
    wid = sid * _SC_NC + cid
    base = wid * _BPW
    pltpu.sync_copy(idx_hbm.at[pl.ds(base, _BPW)], idx_v)
    gather = pltpu.async_copy(table_hbm.at[idx_v], rows_v, sem)

    def ofill(i, _):
        ones_v[pl.ds(i * 16, 16)] = jnp.ones((16,), jnp.float32)
        return 0

    lax.fori_loop(0, _BPW // 16, ofill, 0)

    @pl.when(sid == 0)
    def _():
        pltpu.sync_copy(zeros_hbm, shared_cnt)

    plsc.subcore_barrier()
    pltpu.sync_copy(ones_v, shared_cnt.at[idx_v], add=True)
    gather.wait()
    pltpu.sync_copy(rows_v, out_hbm.at[pl.ds(base, _BPW)])
    plsc.subcore_barrier()

    @pl.when(sid == 0)
    def _():
        pltpu.sync_copy(shared_cnt, cnt_hbm.at[cid])


@functools.cache
def _get_gather_call():
    return pl.kernel(
        _gather_body,
        mesh=plsc.VectorSubcoreMesh(core_axis_name="c", subcore_axis_name="s"),
        out_type=[
            jax.ShapeDtypeStruct((N, D), jnp.float32),
            jax.ShapeDtypeStruct((_SC_NC, K), jnp.float32),
        ],
        scratch_types=[
            pltpu.VMEM((_BPW,), jnp.int32),
            pltpu.VMEM((_BPW, D), jnp.float32),
            pltpu.VMEM((_BPW,), jnp.float32),
            pltpu.VMEM_SHARED((K,), jnp.float32),
            pltpu.SemaphoreType.DMA,
        ],
        compiler_params=pltpu.CompilerParams(use_tc_tiling_on_sc=False),
    )


def _stats_body(z_ref, q_ref, cnt_ref, loss_ref, ppl_ref):
    diff = q_ref[...] - z_ref[...]
    m = jnp.mean(diff * diff)
    loss_ref[...] = jnp.reshape(1.0 * m + 0.25 * m, (1, 1))
    cnt = cnt_ref[...]
    e = (cnt[0:1, :] + cnt[1:2, :]) * (1.0 / N)
    ent = jnp.sum(e * jnp.log(e + 1e-10))
    ppl_ref[...] = jnp.reshape(jnp.exp(-ent), (1, 1))


_stats_call = pl.pallas_call(
    _stats_body,
    out_shape=[
        jax.ShapeDtypeStruct((1, 1), jnp.float32),
        jax.ShapeDtypeStruct((1, 1), jnp.float32),
    ],
)


def kernel(z, emb_weight, W, b):
    zt = jnp.transpose(z, (0, 2, 3, 1))
    z_flat = zt.reshape(-1, D)
    codebook, c2 = _codebook_call(emb_weight, W, b.reshape(1, D))
    ids_f = jnp.arange(K, dtype=jnp.float32).reshape(1, K)
    d, idx3 = _dist_call(z_flat, codebook, c2, ids_f)
    idx = idx3.reshape(N)
    z_q_st, counts = _get_gather_call()(codebook, idx, jnp.zeros((K,), jnp.float32))
    loss11, ppl11 = _stats_call(z_flat, z_q_st, counts)
    z_q_out = jnp.transpose(z_q_st.reshape(zt.shape), (0, 3, 1, 2))
    loss = loss11[0, 0]
    perplexity = ppl11[0, 0]
    perplexity_loss = jnp.zeros((), jnp.float32)
    return (z_q_out, loss, d, perplexity, idx, perplexity_loss)


# speedup vs baseline: 3.3066x; 1.0121x over previous
import functools

import jax
import jax.numpy as jnp
from jax import lax
from jax.experimental import pallas as pl
from jax.experimental.pallas import tpu as pltpu
from jax.experimental.pallas import tpu_sc as plsc

N = 8192
K = 8192
D = 64
TN = 512
TK = K
NI = N // TN

_PREC = lax.Precision.DEFAULT


def _codebook_body(emb_ref, w_ref, b_ref, out_ref, c2_ref):
    cb = lax.dot_general(
        emb_ref[...], w_ref[...], (((1,), (1,)), ((), ())),
        precision=_PREC, preferred_element_type=jnp.float32) + b_ref[...]
    out_ref[...] = cb
    c2_ref[...] = jnp.sum(cb * cb, axis=1).reshape(1, K)


_codebook_call = pl.pallas_call(
    _codebook_body,
    out_shape=[
        jax.ShapeDtypeStruct((K, D), jnp.float32),
        jax.ShapeDtypeStruct((1, K), jnp.float32),
    ],
)


def _dist_body(z_ref, c_ref, c2_ref, ids_ref, d_ref, idx_ref):
    z = z_ref[...]
    c = c_ref[...]
    z2 = jnp.sum(z * z, axis=1, keepdims=True)
    c2 = c2_ref[...]
    dot = lax.dot_general(z, c, (((1,), (1,)), ((), ())),
                          precision=_PREC, preferred_element_type=jnp.float32)
    dist = z2 + c2 - 2.0 * dot
    d_ref[...] = dist
    row_min = jnp.min(dist, axis=1)
    masked = jnp.where(dist == row_min[:, None], ids_ref[...], jnp.float32(2 * K))
    idx_ref[...] = jnp.min(masked, axis=1).astype(jnp.int32).reshape(1, 1, TN)


_dist_call = pl.pallas_call(
    _dist_body,
    grid=(NI,),
    in_specs=[
        pl.BlockSpec((TN, D), lambda i: (i, 0)),
        pl.BlockSpec((K, D), lambda i: (0, 0)),
        pl.BlockSpec((1, K), lambda i: (0, 0)),
        pl.BlockSpec((1, K), lambda i: (0, 0)),
    ],
    out_specs=[
        pl.BlockSpec((TN, K), lambda i: (i, 0)),
        pl.BlockSpec((1, 1, TN), lambda i: (i, 0, 0)),
    ],
    out_shape=[
        jax.ShapeDtypeStruct((N, K), jnp.float32),
        jax.ShapeDtypeStruct((NI, 1, TN), jnp.int32),
    ],
)


_SC_NC = 2
_SC_NS = 16
_NW = _SC_NC * _SC_NS
_BPW = N // _NW


def _gather_body(table_hbm, idx_hbm, zeros_hbm, out_hbm, cnt_hbm,
                 idx_v, rows_v, ones_v, shared_cnt, sem):
    cid = lax.axis_index("c")
    sid = lax.axis_index("s")
    wid = sid * _SC_NC + cid
    base = wid * _BPW
    pltpu.sync_copy(idx_hbm.at[pl.ds(base, _BPW)], idx_v)
    gather = pltpu.async_copy(table_hbm.at[idx_v], rows_v, sem)

    def ofill(i, _):
        ones_v[pl.ds(i * 16, 16)] = jnp.ones((16,), jnp.float32)
        return 0

    lax.fori_loop(0, _BPW // 16, ofill, 0)

    @pl.when(sid == 0)
    def _():
        pltpu.sync_copy(zeros_hbm, shared_cnt)

    plsc.subcore_barrier()
    pltpu.sync_copy(ones_v, shared_cnt.at[idx_v], add=True)
    gather.wait()
    pltpu.sync_copy(rows_v, out_hbm.at[pl.ds(base, _BPW)])
    plsc.subcore_barrier()

    @pl.when(sid == 0)
    def _():
        pltpu.sync_copy(shared_cnt, cnt_hbm.at[cid])


@functools.cache
def _get_gather_call():
    return pl.kernel(
        _gather_body,
        mesh=plsc.VectorSubcoreMesh(core_axis_name="c", subcore_axis_name="s"),
        out_type=[
            jax.ShapeDtypeStruct((N, D), jnp.float32),
            jax.ShapeDtypeStruct((_SC_NC, K), jnp.float32),
        ],
        scratch_types=[
            pltpu.VMEM((_BPW,), jnp.int32),
            pltpu.VMEM((_BPW, D), jnp.float32),
            pltpu.VMEM((_BPW,), jnp.float32),
            pltpu.VMEM_SHARED((K,), jnp.float32),
            pltpu.SemaphoreType.DMA,
        ],
        compiler_params=pltpu.CompilerParams(use_tc_tiling_on_sc=False),
    )


def _stats_body(z_ref, q_ref, cnt_ref, loss_ref, ppl_ref):
    diff = q_ref[...] - z_ref[...]
    m = jnp.mean(diff * diff)
    loss_ref[...] = jnp.reshape(1.0 * m + 0.25 * m, (1, 1))
    cnt = cnt_ref[...]
    e = (cnt[0:1, :] + cnt[1:2, :]) * (1.0 / N)
    ent = jnp.sum(e * jnp.log(e + 1e-10))
    ppl_ref[...] = jnp.reshape(jnp.exp(-ent), (1, 1))


_stats_call = pl.pallas_call(
    _stats_body,
    out_shape=[
        jax.ShapeDtypeStruct((1, 1), jnp.float32),
        jax.ShapeDtypeStruct((1, 1), jnp.float32),
    ],
)


def kernel(z, emb_weight, W, b):
    zt = jnp.transpose(z, (0, 2, 3, 1))
    z_flat = zt.reshape(-1, D)
    codebook, c2 = _codebook_call(emb_weight, W, b.reshape(1, D))
    ids_f = jnp.arange(K, dtype=jnp.float32).reshape(1, K)
    d, idx3 = _dist_call(z_flat, codebook, c2, ids_f)
    idx = idx3.reshape(N)
    z_q_st, counts = _get_gather_call()(codebook, idx, jnp.zeros((K,), jnp.float32))
    loss11, ppl11 = _stats_call(z_flat, z_q_st, counts)
    z_q_out = jnp.transpose(z_q_st.reshape(zt.shape), (0, 3, 1, 2))
    loss = loss11[0, 0]
    perplexity = ppl11[0, 0]
    perplexity_loss = jnp.zeros((), jnp.float32)
    return (z_q_out, loss, d, perplexity, idx, perplexity_loss)


# speedup vs baseline: 3.3706x; 1.0193x over previous
import functools

import jax
import jax.numpy as jnp
from jax import lax
from jax.experimental import pallas as pl
from jax.experimental.pallas import tpu as pltpu
from jax.experimental.pallas import tpu_sc as plsc

N = 8192
K = 8192
D = 64
TN = 512
TK = K
NI = N // TN

_PREC = lax.Precision.DEFAULT


def _dist_body(emb_ref, w_ref, b_ref, z_ref, d_ref, idx_ref, cb_ref,
               c2_ref, ids_ref):
    i = pl.program_id(0)

    @pl.when(i == 0)
    def _():
        cb0 = lax.dot_general(
            emb_ref[...], w_ref[...], (((1,), (1,)), ((), ())),
            precision=_PREC, preferred_element_type=jnp.float32) + b_ref[...]
        cb_ref[...] = cb0
        c2_ref[...] = jnp.sum(cb0 * cb0, axis=1).reshape(1, K)
        ids_ref[...] = lax.broadcasted_iota(jnp.int32, (1, K), 1).astype(jnp.float32)

    z = z_ref[...]
    c = cb_ref[...]
    z2 = jnp.sum(z * z, axis=1, keepdims=True)
    c2 = c2_ref[...]
    dot = lax.dot_general(z, c, (((1,), (1,)), ((), ())),
                          precision=_PREC, preferred_element_type=jnp.float32)
    dist = z2 + c2 - 2.0 * dot
    d_ref[...] = dist
    row_min = jnp.min(dist, axis=1)
    masked = jnp.where(dist == row_min[:, None], ids_ref[...], jnp.float32(2 * K))
    idx_ref[...] = jnp.min(masked, axis=1).astype(jnp.int32).reshape(1, 1, TN)


_dist_call = pl.pallas_call(
    _dist_body,
    grid=(NI,),
    in_specs=[
        pl.BlockSpec((K, D), lambda i: (0, 0)),
        pl.BlockSpec((D, D), lambda i: (0, 0)),
        pl.BlockSpec((1, D), lambda i: (0, 0)),
        pl.BlockSpec((TN, D), lambda i: (i, 0)),
    ],
    out_specs=[
        pl.BlockSpec((TN, K), lambda i: (i, 0)),
        pl.BlockSpec((1, 1, TN), lambda i: (i, 0, 0)),
        pl.BlockSpec((K, D), lambda i: (0, 0)),
    ],
    out_shape=[
        jax.ShapeDtypeStruct((N, K), jnp.float32),
        jax.ShapeDtypeStruct((NI, 1, TN), jnp.int32),
        jax.ShapeDtypeStruct((K, D), jnp.float32),
    ],
    scratch_shapes=[
        pltpu.VMEM((1, K), jnp.float32),
        pltpu.VMEM((1, K), jnp.float32),
    ],
)


_SC_NC = 2
_SC_NS = 16
_NW = _SC_NC * _SC_NS
_BPW = N // _NW


def _gather_body(table_hbm, idx_hbm, zeros_hbm, out_hbm, cnt_hbm,
                 idx_v, rows_v, ones_v, shared_cnt, sem):
    cid = lax.axis_index("c")
    sid = lax.axis_index("s")
    wid = sid * _SC_NC + cid
    base = wid * _BPW
    pltpu.sync_copy(idx_hbm.at[pl.ds(base, _BPW)], idx_v)
    gather = pltpu.async_copy(table_hbm.at[idx_v], rows_v, sem)

    def ofill(i, _):
        ones_v[pl.ds(i * 16, 16)] = jnp.ones((16,), jnp.float32)
        return 0

    lax.fori_loop(0, _BPW // 16, ofill, 0)

    @pl.when(sid == 0)
    def _():
        pltpu.sync_copy(zeros_hbm, shared_cnt)

    plsc.subcore_barrier()
    pltpu.sync_copy(ones_v, shared_cnt.at[idx_v], add=True)
    gather.wait()
    pltpu.sync_copy(rows_v, out_hbm.at[pl.ds(base, _BPW)])
    plsc.subcore_barrier()

    @pl.when(sid == 0)
    def _():
        pltpu.sync_copy(shared_cnt, cnt_hbm.at[cid])


@functools.cache
def _get_gather_call():
    return pl.kernel(
        _gather_body,
        mesh=plsc.VectorSubcoreMesh(core_axis_name="c", subcore_axis_name="s"),
        out_type=[
            jax.ShapeDtypeStruct((N, D), jnp.float32),
            jax.ShapeDtypeStruct((_SC_NC, K), jnp.float32),
        ],
        scratch_types=[
            pltpu.VMEM((_BPW,), jnp.int32),
            pltpu.VMEM((_BPW, D), jnp.float32),
            pltpu.VMEM((_BPW,), jnp.float32),
            pltpu.VMEM_SHARED((K,), jnp.float32),
            pltpu.SemaphoreType.DMA,
        ],
        compiler_params=pltpu.CompilerParams(use_tc_tiling_on_sc=False),
    )


def _stats_body(z_ref, q_ref, cnt_ref, loss_ref, ppl_ref):
    diff = q_ref[...] - z_ref[...]
    m = jnp.mean(diff * diff)
    loss_ref[...] = jnp.reshape(1.0 * m + 0.25 * m, (1, 1))
    cnt = cnt_ref[...]
    e = (cnt[0:1, :] + cnt[1:2, :]) * (1.0 / N)
    ent = jnp.sum(e * jnp.log(e + 1e-10))
    ppl_ref[...] = jnp.reshape(jnp.exp(-ent), (1, 1))


_stats_call = pl.pallas_call(
    _stats_body,
    out_shape=[
        jax.ShapeDtypeStruct((1, 1), jnp.float32),
        jax.ShapeDtypeStruct((1, 1), jnp.float32),
    ],
)


def kernel(z, emb_weight, W, b):
    zt = jnp.transpose(z, (0, 2, 3, 1))
    z_flat = zt.reshape(-1, D)
    d, idx3, codebook = _dist_call(emb_weight, W, b.reshape(1, D), z_flat)
    idx = idx3.reshape(N)
    z_q_st, counts = _get_gather_call()(codebook, idx, jnp.zeros((K,), jnp.float32))
    loss11, ppl11 = _stats_call(z_flat, z_q_st, counts)
    z_q_out = jnp.transpose(z_q_st.reshape(zt.shape), (0, 3, 1, 2))
    loss = loss11[0, 0]
    perplexity = ppl11[0, 0]
    perplexity_loss = jnp.zeros((), jnp.float32)
    return (z_q_out, loss, d, perplexity, idx, perplexity_loss)
